# baseline (speedup 1.0000x reference)
import jax
import jax.numpy as jnp
from jax.experimental import pallas as pl

DIM = 96; RANK = 16; E = 4; TOPK = 1; PS = 8


def _conv1x1(x, w, b=None):
    y = jnp.einsum('bchw,oc->bohw', x, w)
    if b is not None:
        y = y + b[None, :, None, None]
    return y


def _dwconv(x, w, b, pad):
    c = x.shape[1]
    y = jax.lax.conv_general_dilated(x, w, (1, 1), [(pad, pad), (pad, pad)],
                                     dimension_numbers=('NCHW', 'OIHW', 'NCHW'),
                                     feature_group_count=c)
    return y + b[None, :, None, None]


def _layernorm_c(x, w, b):
    mu = jnp.mean(x, axis=1, keepdims=True)
    var = jnp.mean((x - mu) ** 2, axis=1, keepdims=True)
    return (x - mu) / jnp.sqrt(var + 1e-5) * w[None, :, None, None] + b[None, :, None, None]


def _fft_attn(x, qw, qdw, qdb, kvw, kvdw, kvdb, lnw, lnb, ow, ob):
    b, c, h, w = x.shape
    q = _dwconv(_conv1x1(x, qw), qdw, qdb, 1)
    kv = _dwconv(_conv1x1(x, kvw), kvdw, kvdb, 3)
    k, v = jnp.split(kv, 2, axis=1)
    ph = (-h) % PS; pw = (-w) % PS
    def par(t):
        t = jnp.pad(t, ((0, 0), (0, 0), (0, ph), (0, pw)))
        return t.reshape(b, c, (h + ph) // PS, PS, (w + pw) // PS, PS).transpose(0, 1, 2, 4, 3, 5)
    o = jnp.fft.irfft2(jnp.fft.rfft2(par(q)) * jnp.fft.rfft2(par(k)), s=(PS, PS))
    bb, cc, hh, ww, _, _ = o.shape
    o = o.transpose(0, 1, 2, 4, 3, 5).reshape(bb, cc, hh * PS, ww * PS)[:, :, :h, :w]
    o = _layernorm_c(o, lnw, lnb)
    o = o * v
    return _conv1x1(o, ow, ob)


def _expert_fwd(x, shared, p0, p1, p2, qw, qdw, qdb, kvw, kvdw, kvdb, lnw, lnb, ow, ob):
    h = _conv1x1(x, p0)
    a = _fft_attn(h, qw, qdw, qdb, kvw, kvdw, kvdb, lnw, lnb, ow, ob)
    g = jax.nn.silu(_conv1x1(shared, p1))
    return _conv1x1(a * g, p2) + x


def _router_fn(x, rw, rb):
    probs = jax.nn.softmax(_conv1x1(x, rw, rb), axis=1)
    pl_ = jnp.moveaxis(probs, 1, -1)
    vals, idx = jax.lax.top_k(pl_, TOPK)
    gl = jnp.sum(jax.nn.one_hot(idx, E, dtype=probs.dtype) * vals[..., None], axis=-2)
    return jnp.moveaxis(gl, -1, 1)


def _projout_kernel(x_ref, w_ref, b_ref, o_ref):
    o_ref[...] = (jnp.dot(w_ref[...], x_ref[...],
                          preferred_element_type=jnp.float32)
                  + b_ref[...].reshape(DIM, 1))


def _projout_pallas(xflat, w, b):
    HW = xflat.shape[1]
    T = 9216
    grid = (HW // T,)
    return pl.pallas_call(
        _projout_kernel,
        grid=grid,
        in_specs=[
            pl.BlockSpec((DIM, T), lambda i: (0, i)),
            pl.BlockSpec((DIM, DIM), lambda i: (0, 0)),
            pl.BlockSpec((DIM,), lambda i: (0,)),
        ],
        out_specs=pl.BlockSpec((DIM, T), lambda i: (0, i)),
        out_shape=jax.ShapeDtypeStruct((DIM, HW), jnp.float32),
    )(xflat, w, b)


def kernel(x, shared, router_w, router_b, e_p0, e_p1, e_p2, e_qw, e_qdw, e_qdb,
           e_kvw, e_kvdw, e_kvdb, e_lnw, e_lnb, e_pow, e_pob, projout_w, projout_b):
    gates = _router_fn(x, router_w, router_b)
    out = jnp.zeros_like(x)
    for i in range(E):
        ex = _expert_fwd(x, shared, e_p0[i], e_p1[i], e_p2[i], e_qw[i], e_qdw[i],
                         e_qdb[i], e_kvw[i], e_kvdw[i], e_kvdb[i], e_lnw[i],
                         e_lnb[i], e_pow[i], e_pob[i])
        out = out + ex * gates[:, i:i + 1]
    b, c, h, w = out.shape
    y = _projout_pallas(out.reshape(c, h * w), projout_w, projout_b)
    return y.reshape(b, c, h, w)


# speedup vs baseline: 2.4259x; 2.4259x over previous
import jax
import jax.numpy as jnp
from jax.experimental import pallas as pl
from jax.experimental.pallas import tpu as pltpu

DIM = 96
RANK = 16
E = 4
PS = 8
H = 384
W = 384
HB = 8
NB = H // HB
HALO = 3
HEXT = HB + 2 * HALO
NPX = HB * W

KVR = 2 * RANK
QC = E * RANK
KVC = E * KVR
PRE_C = QC + KVC + E


def _band_kernel(xe_ref, sh_ref, wpre_ref, rb_ref, qdw_ref, qdb_ref,
                 kvdw_ref, kvdb_ref, lnw_ref, lnb_ref, ow_ref, ob_ref,
                 p1_ref, p2_ref, pw_ref, pb_ref, out_ref, pre_ref):
    f32 = jnp.float32
    e = pl.program_id(1)
    xe = xe_ref[0]

    @pl.when(e == 0)
    def _():
        xm = xe.reshape(DIM, HEXT * W)
        pre_ref[...] = jnp.dot(wpre_ref[...], xm,
                               preferred_element_type=f32
                               ).reshape(PRE_C, HEXT, W)

    logits = pre_ref[QC + KVC:, HALO:HALO + HB, :] + rb_ref[0][:, None, None]
    lmax = jnp.max(logits, axis=0, keepdims=True)
    exl = jnp.exp(logits - lmax)
    probs = exl / jnp.sum(exl, axis=0, keepdims=True)
    pmax = jnp.max(probs, axis=0)
    hit0 = probs[0] == pmax
    hit1 = probs[1] == pmax
    hit2 = probs[2] == pmax
    idx = jnp.where(hit0, 0, jnp.where(hit1, 1, jnp.where(hit2, 2, 3)))
    gate = jnp.where(idx == e, pmax, 0.0)

    qpre = pre_ref[pl.ds(e * RANK, RANK)]
    qprep = jnp.pad(qpre, ((0, 0), (0, 0), (HALO, HALO)))
    q3 = jnp.zeros((RANK, HB, W), f32)
    for dy in range(3):
        for dx in range(3):
            wtap = qdw_ref[0, :, 3 * dy + dx][:, None, None]
            q3 = q3 + wtap * qprep[:, 2 + dy:2 + dy + HB, 2 + dx:2 + dx + W]
    q3 = q3 + qdb_ref[0, 0][:, None, None]

    kvpre = pre_ref[pl.ds(QC + e * KVR, KVR)]
    kvprep = jnp.pad(kvpre, ((0, 0), (0, 0), (HALO, HALO)))
    k7 = jnp.zeros((KVR, HB, W), f32)
    for dy in range(7):
        for dx in range(7):
            wtap = kvdw_ref[0, :, 7 * dy + dx][:, None, None]
            k7 = k7 + wtap * kvprep[:, dy:dy + HB, dx:dx + W]
    k7 = k7 + kvdb_ref[0, 0][:, None, None]
    kblk = k7[:RANK]
    vblk = k7[RANK:]

    lane = jax.lax.broadcasted_iota(jnp.int32, (1, 1, W), 2)
    loff = lane % PS
    o = jnp.zeros((RANK, HB, W), f32)
    kt = kblk
    qa = q3
    for t in range(PS):
        if t > 0:
            kt = jnp.where(loff >= t, jnp.roll(kt, 1, axis=2),
                           jnp.roll(kt, 1 - PS, axis=2))
            qa = jnp.roll(qa, -1, axis=2)
        qb = jnp.where(loff == 0, qa, 0.0)
        qb = qb + jnp.roll(qb, 1, axis=2)
        qb = qb + jnp.roll(qb, 2, axis=2)
        qb = qb + jnp.roll(qb, 4, axis=2)
        krs = kt
        for s in range(PS):
            if s > 0:
                krs = jnp.roll(krs, 1, axis=1)
            o = o + qb[:, s:s + 1, :] * krs

    om = o.reshape(RANK, NPX)
    mu = jnp.mean(om, axis=0, keepdims=True)
    var = jnp.mean((om - mu) ** 2, axis=0, keepdims=True)
    oln = (om - mu) / jnp.sqrt(var + 1e-5)
    oln = oln * lnw_ref[0, 0][:, None] + lnb_ref[0, 0][:, None]
    ov = oln * vblk.reshape(RANK, NPX)
    a = jnp.dot(ow_ref[0], ov, preferred_element_type=f32) \
        + ob_ref[0, 0][:, None]

    shm = sh_ref[...].reshape(DIM, NPX)
    galin = jnp.dot(p1_ref[0], shm, preferred_element_type=f32)
    g = galin * jax.nn.sigmoid(galin)
    ag = (a * g) * gate.reshape(1, NPX)
    core = jnp.dot(p2_ref[0], ag, preferred_element_type=f32)
    contrib = core.reshape(DIM, HB, W) + gate[None] * xe[:, HALO:HALO + HB, :]

    @pl.when(e == 0)
    def _():
        out_ref[...] = contrib

    @pl.when(e > 0)
    def _():
        out_ref[...] += contrib

    @pl.when(e == E - 1)
    def _():
        acc = out_ref[...].reshape(DIM, NPX)
        y = jnp.dot(pw_ref[...], acc, preferred_element_type=f32) \
            + pb_ref[0][:, None]
        out_ref[...] = y.reshape(DIM, HB, W)


def _shared(shape):
    nd = len(shape)
    return pl.BlockSpec(shape, lambda i, e, _n=nd: (0,) * _n)


def _perexp(shape):
    nd = len(shape)
    return pl.BlockSpec((1,) + shape[1:],
                        lambda i, e, _n=nd: (e,) + (0,) * (_n - 1))


def kernel(x, shared, router_w, router_b, e_p0, e_p1, e_p2, e_qw, e_qdw,
           e_qdb, e_kvw, e_kvdw, e_kvdb, e_lnw, e_lnb, e_pow, e_pob,
           projout_w, projout_b):
    f32 = jnp.float32
    wq = jnp.einsum('eij,ejc->eic', e_qw, e_p0).reshape(QC, DIM)
    wkv = jnp.einsum('eij,ejc->eic', e_kvw, e_p0).reshape(KVC, DIM)
    wpre = jnp.concatenate([wq, wkv, router_w], axis=0)
    qdw = e_qdw.reshape(E, RANK, 9)
    kvdw = e_kvdw.reshape(E, KVR, 49)
    xpad = jnp.pad(x[0], ((0, 0), (HALO, HALO), (0, 0)))
    xext = jnp.stack([xpad[:, i * HB:i * HB + HEXT, :] for i in range(NB)])

    out = pl.pallas_call(
        _band_kernel,
        grid=(NB, E),
        in_specs=[
            pl.BlockSpec((1, DIM, HEXT, W), lambda i, e: (i, 0, 0, 0)),
            pl.BlockSpec((DIM, HB, W), lambda i, e: (0, i, 0)),
            _shared((PRE_C, DIM)),
            _shared((1, E)),
            _perexp((E, RANK, 9)),
            _perexp((E, 1, RANK)),
            _perexp((E, KVR, 49)),
            _perexp((E, 1, KVR)),
            _perexp((E, 1, RANK)),
            _perexp((E, 1, RANK)),
            _perexp((E, RANK, RANK)),
            _perexp((E, 1, RANK)),
            _perexp((E, RANK, DIM)),
            _perexp((E, DIM, RANK)),
            _shared((DIM, DIM)),
            _shared((1, DIM)),
        ],
        out_specs=pl.BlockSpec((DIM, HB, W), lambda i, e: (0, i, 0)),
        out_shape=jax.ShapeDtypeStruct((DIM, H, W), f32),
        scratch_shapes=[pltpu.VMEM((PRE_C, HEXT, W), f32)],
    )(xext, shared[0], wpre, router_b.reshape(1, E), qdw,
      e_qdb.reshape(E, 1, RANK), kvdw, e_kvdb.reshape(E, 1, KVR),
      e_lnw.reshape(E, 1, RANK), e_lnb.reshape(E, 1, RANK), e_pow,
      e_pob.reshape(E, 1, RANK), e_p1, e_p2, projout_w,
      projout_b.reshape(1, DIM))
    return out[None].astype(f32)
